# Initial kernel scaffold; baseline (speedup 1.0000x reference)
import jax
import jax.numpy as jnp
from jax.experimental import pallas as pl


def kernel(cate, cat_incre, obs, target, time, emb_mu, emb_logvar, itemw_mu, t_W1, t_b1, t_W2, t_b2, fc_W, fc_b):
    raise NotImplementedError("write your pallas kernel here")


# speedup vs baseline: 33.2801x; 33.2801x over previous
import functools

import jax
import jax.numpy as jnp
from jax import lax
from jax.experimental import pallas as pl
from jax.experimental.pallas import tpu as pltpu
from jax.experimental.pallas import tpu_sc as plsc

VOCAB = 100000
LATENT = 32
NUM_DISC = 26
NUM_CONT = 16
HID = 64

_NC = 2
_NS = 16
_NW = _NC * _NS

_GATHER_ROWS = 128
_GROUP = 8
_CHUNK = _GATHER_ROWS * _GROUP


def _sc_gather(table, idx2d, total_rows):
    per_w = total_rows // _NW
    chunks = per_w // _CHUNK
    mesh = plsc.VectorSubcoreMesh(core_axis_name="c", subcore_axis_name="s")

    @functools.partial(
        pl.kernel,
        mesh=mesh,
        out_type=jax.ShapeDtypeStruct((total_rows, 32), jnp.float32),
        scratch_types=[
            pltpu.VMEM((_GROUP, _GATHER_ROWS), jnp.int32),
            pltpu.VMEM((_CHUNK, 32), jnp.float32),
            pltpu.SemaphoreType.DMA,
        ],
        compiler_params=pltpu.CompilerParams(use_tc_tiling_on_sc=False),
    )
    def k(table_hbm, idx_hbm, out_hbm, idx_v, rows_v, sem):
        wid = lax.axis_index("s") * _NC + lax.axis_index("c")

        def body(i, _):
            chunk_id = wid * chunks + i
            pltpu.sync_copy(idx_hbm.at[pl.ds(chunk_id * _GROUP, _GROUP)], idx_v)
            descs = [
                pltpu.async_copy(
                    table_hbm.at[idx_v.at[j]],
                    rows_v.at[pl.ds(j * _GATHER_ROWS, _GATHER_ROWS)],
                    sem,
                )
                for j in range(_GROUP)
            ]
            for d in descs:
                d.wait()
            pltpu.sync_copy(rows_v, out_hbm.at[pl.ds(chunk_id * _CHUNK, _CHUNK)])
            return _

        lax.fori_loop(0, chunks, body, None)

    return k(table, idx2d)


def _tc_head(g, aux, Wc, Waux, W2, b1, b2, fct, fcb):
    N = g.shape[0]
    T = 1024
    grid = (N // T,)

    def body(g_ref, aux_ref, wc_ref, waux_ref, w2_ref, b1_ref, b2_ref,
             fct_ref, fcb_ref, rate_ref, tmu_ref, tsd_ref):
        u = jnp.dot(g_ref[...], wc_ref[...], preferred_element_type=jnp.float32)
        u = u + jnp.dot(aux_ref[...], waux_ref[...],
                        preferred_element_type=jnp.float32)
        h = jnp.tanh(u[:, :HID] + b1_ref[...])
        t_out = jnp.dot(h, w2_ref[...], preferred_element_type=jnp.float32)
        t_out = t_out + b2_ref[...]
        t_mu = t_out[:, :LATENT]
        tmu_ref[...] = t_mu
        tsd_ref[...] = jnp.exp(0.5 * t_out[:, LATENT:])
        v = (u[:, HID:HID + 1]
             + jnp.sum(t_mu * fct_ref[...], axis=1, keepdims=True)
             + fcb_ref[...])
        rate_ref[...] = jnp.exp(v)

    return pl.pallas_call(
        body,
        grid=grid,
        in_specs=[
            pl.BlockSpec((T, 832), lambda i: (i, 0)),
            pl.BlockSpec((T, 18), lambda i: (i, 0)),
            pl.BlockSpec((832, 65), lambda i: (0, 0)),
            pl.BlockSpec((18, 65), lambda i: (0, 0)),
            pl.BlockSpec((HID, HID), lambda i: (0, 0)),
            pl.BlockSpec((1, HID), lambda i: (0, 0)),
            pl.BlockSpec((1, HID), lambda i: (0, 0)),
            pl.BlockSpec((1, LATENT), lambda i: (0, 0)),
            pl.BlockSpec((1, 1), lambda i: (0, 0)),
        ],
        out_specs=[
            pl.BlockSpec((T, 1), lambda i: (i, 0)),
            pl.BlockSpec((T, LATENT), lambda i: (i, 0)),
            pl.BlockSpec((T, LATENT), lambda i: (i, 0)),
        ],
        out_shape=[
            jax.ShapeDtypeStruct((N, 1), jnp.float32),
            jax.ShapeDtypeStruct((N, LATENT), jnp.float32),
            jax.ShapeDtypeStruct((N, LATENT), jnp.float32),
        ],
        compiler_params=pltpu.CompilerParams(
            dimension_semantics=("parallel",),
        ),
    )(g, aux, Wc, Waux, W2, b1, b2, fct, fcb)


def kernel(cate, cat_incre, obs, target, time, emb_mu, emb_logvar, itemw_mu,
           t_W1, t_b1, t_W2, t_b2, fc_W, fc_b):
    B, S = cate.shape[0], cate.shape[1]
    N = B * S
    EMB = NUM_DISC * LATENT

    idx = jnp.concatenate(
        [cate.reshape(N, 20), cat_incre.reshape(N, 6)], axis=1
    ).astype(jnp.int32)
    total_rows = N * NUM_DISC
    idx2d = idx.reshape(total_rows // _GATHER_ROWS, _GATHER_ROWS)

    g = _sc_gather(emb_mu, idx2d, total_rows)
    g = g.reshape(N, EMB)

    aux = jnp.concatenate(
        [obs.reshape(N, NUM_CONT), target.reshape(N, 1), time.reshape(N, 1)],
        axis=1)
    Wc = jnp.concatenate([t_W1[:EMB], fc_W[:EMB]], axis=1)
    Waux = jnp.concatenate(
        [t_W1[EMB:EMB + NUM_CONT + 2],
         jnp.concatenate([fc_W[EMB + LATENT:], jnp.zeros((2, 1), jnp.float32)],
                         axis=0)],
        axis=1)
    fct = fc_W[EMB:EMB + LATENT].reshape(1, LATENT)

    rate2d, t_mu, t_sd = _tc_head(
        g, aux, Wc, Waux, t_W2,
        t_b1.reshape(1, HID), t_b2.reshape(1, 2 * LATENT),
        fct, fc_b.reshape(1, 1))

    rate = rate2d.reshape(B, S)
    t_mu = t_mu.reshape(B, S, LATENT)
    t_sd = t_sd.reshape(B, S, LATENT)

    q_item = (emb_mu, jnp.exp(0.5 * emb_logvar))
    q_itemw = (itemw_mu, jnp.ones_like(itemw_mu))
    q_time = (t_mu, t_sd)
    p_item = (jnp.zeros_like(emb_mu), jnp.ones_like(emb_mu))
    p_itemw = (jnp.zeros_like(itemw_mu), jnp.ones_like(itemw_mu))
    p_time = (jnp.zeros_like(t_mu), jnp.ones_like(t_mu))
    return (rate, q_item, q_itemw, q_time, p_item, p_itemw, p_time)
